# baseline (speedup 1.0000x reference)
import jax
import jax.numpy as jnp
from jax.experimental import pallas as pl

B = 64
N = 64
D = 64
OUT = 3
OUT_PAD = 128
ALPHA0 = 0.1
ALPHA1 = 0.1


def _lrelu(v, a):
    return jnp.maximum(v, a * v)


def _dot(a, b):
    return jnp.dot(a.astype(jnp.bfloat16), b.astype(jnp.bfloat16),
                   preferred_element_type=jnp.float32)


def _bf(v):
    return v.astype(jnp.bfloat16).astype(jnp.float32)


def _proj_kernel(x_ref, w_ref, b_ref, out_ref):
    out_ref[...] = (
        _dot(x_ref[...], w_ref[...])
        + b_ref[...]
    )


def _mp_block(h, Msel, Wi, Wj, wd, eb, W1, b1, n0a, n0b, nb0, n1, nb1, n2,
              nb2, alpha, last):
    A = _dot(h, Wi)
    Bm = _dot(h, Wj)
    hh = h * h
    G = jax.lax.dot_general(h, h, (((1,), (1,)), ((), ())),
                            preferred_element_type=jnp.float32,
                            precision=jax.lax.Precision.HIGHEST)
    r = jnp.sum(hh, axis=1, keepdims=True)
    ones_row = jnp.ones((1, N), jnp.float32)
    rT = jax.lax.dot_general(ones_row, hh, (((1,), (1,)), ((), ())),
                             preferred_element_type=jnp.float32,
                             precision=jax.lax.Precision.HIGHEST)
    dist = r + rT - 2.0 * G
    A2 = A + eb
    pre = (A2[:, None, :] + Bm[None, :, :]
           + _bf(dist)[:, :, None] * _bf(wd)[None, :, :])
    e = _lrelu(pre, alpha)
    e2 = _lrelu(
        _dot(e.reshape(N * N, 64), W1)
        + b1, alpha)
    agg = jax.lax.dot_general(Msel, e2, (((1,), (0,)), ((), ())),
                              preferred_element_type=jnp.float32,
                              precision=jax.lax.Precision.HIGHEST)
    n = _lrelu(_dot(h, n0a) + _dot(agg, n0b) + nb0, alpha)
    n = _lrelu(_dot(n, n1) + nb1, alpha)
    n = _dot(n, n2) + nb2
    if not last:
        n = _lrelu(n, alpha)
    return n


def _mp_kernel(h_ref, msel_ref,
               wi0, wj0, wd0, e0b, w01, b01, na0, nb0_, nbi0, n01, nbi01,
               n02, nbi02,
               wi1, wj1, wd1, e1b, w11, b11, na1, nb1_, nbi1, n11, nbi11,
               n12, nbi12,
               out_ref):
    h = h_ref[0]
    Msel = msel_ref[...]
    h1 = _mp_block(h, Msel, wi0[...], wj0[...], wd0[...], e0b[...], w01[...],
                   b01[...], na0[...], nb0_[...], nbi0[...], n01[...],
                   nbi01[...], n02[...], nbi02[...], ALPHA0, False)
    out = _mp_block(h1, Msel, wi1[...], wj1[...], wd1[...], e1b[...],
                    w11[...], b11[...], na1[...], nb1_[...], nbi1[...],
                    n11[...], nbi11[...], n12[...], nbi12[...], ALPHA1, True)
    out_ref[0] = out


def kernel(x, W0, b0, ew0_0, eb0_0, ew0_1, eb0_1, nw0_0, nb0_0, nw0_1, nb0_1,
           nw0_2, nb0_2, ew1_0, eb1_0, ew1_1, eb1_1, nw1_0, nb1_0, nw1_1,
           nb1_1, nw1_2, nb1_2):
    H = pl.pallas_call(
        _proj_kernel,
        out_shape=jax.ShapeDtypeStruct((B, N * D), jnp.float32),
    )(x, W0, b0.reshape(1, N * D))
    h0 = H.reshape(B, N, D)

    def split_edge(ew):
        return ew[:D], ew[D:2 * D], ew[2 * D:2 * D + 1]

    wi0, wj0, wd0 = split_edge(ew0_0)
    wi1, wj1, wd1 = split_edge(ew1_0)
    na0, nb0h = nw0_0[:D], nw0_0[D:]
    na1, nb1h = nw1_0[:D], nw1_0[D:]
    n12p = jnp.pad(nw1_2, ((0, 0), (0, OUT_PAD - OUT)))
    nbi12p = jnp.pad(nb1_2, (0, OUT_PAD - OUT)).reshape(1, OUT_PAD)

    msel = jnp.kron(jnp.eye(N, dtype=jnp.float32),
                    jnp.ones((1, N), jnp.float32))

    row = lambda v: v.reshape(1, -1)
    weights = (
        wi0, wj0, wd0, row(eb0_0), ew0_1, row(eb0_1), na0, nb0h, row(nb0_0),
        nw0_1, row(nb0_1), nw0_2, row(nb0_2),
        wi1, wj1, wd1, row(eb1_0), ew1_1, row(eb1_1), na1, nb1h, row(nb1_0),
        nw1_1, row(nb1_1), n12p, nbi12p,
    )

    wspecs = [pl.BlockSpec(w.shape, lambda b, nd=w.ndim: (0,) * nd)
              for w in weights]

    out = pl.pallas_call(
        _mp_kernel,
        grid=(B,),
        in_specs=([pl.BlockSpec((1, N, D), lambda b: (b, 0, 0)),
                   pl.BlockSpec((N, N * N), lambda b: (0, 0))] + wspecs),
        out_specs=pl.BlockSpec((1, N, OUT_PAD), lambda b: (b, 0, 0)),
        out_shape=jax.ShapeDtypeStruct((B, N, OUT_PAD), jnp.float32),
    )(h0, msel, *weights)

    return out[:, :, :OUT]


# speedup vs baseline: 2.1137x; 2.1137x over previous
import jax
import jax.numpy as jnp
from jax.experimental import pallas as pl

B = 64
N = 64
D = 64
OUT = 3
OUT_PAD = 128
ALPHA0 = 0.1
ALPHA1 = 0.1


def _lrelu(v, a):
    return jnp.maximum(v, a * v)


def _dot(a, b):
    return jnp.dot(a.astype(jnp.bfloat16), b.astype(jnp.bfloat16),
                   preferred_element_type=jnp.float32)


def _bf(v):
    return v.astype(jnp.bfloat16).astype(jnp.float32)


def _proj_kernel(x_ref, w_ref, b_ref, out_ref):
    out_ref[...] = _dot(x_ref[...], w_ref[...]) + b_ref[...]


def _bf_dist(h):
    hh = h * h
    G = jax.lax.dot_general(h, h, (((1,), (1,)), ((), ())),
                            preferred_element_type=jnp.float32,
                            precision=jax.lax.Precision.HIGHEST)
    r = jnp.sum(hh, axis=1, keepdims=True)
    ones_row = jnp.ones((1, N), jnp.float32)
    rT = jax.lax.dot_general(ones_row, hh, (((1,), (1,)), ((), ())),
                             preferred_element_type=jnp.float32,
                             precision=jax.lax.Precision.HIGHEST)
    return _bf(r + rT - 2.0 * G)


def _pack(v):
    return jnp.concatenate([v[:N], v[N:]], axis=1)


def _mp_block(h_rows, Wi, Wj, wdlo, wdhi, ebT, W1bd, b1T, n0a, n0b, nb0,
              n1, nb1, n2, nb2, alpha, last):
    A_p = _pack(_dot(h_rows, Wi) + ebT[:, :D])
    Bm_p = _pack(_dot(h_rows, Wj))
    dA = _bf_dist(h_rows[:N])
    dB = _bf_dist(h_rows[N:])
    pre = (A_p[:, None, :] + Bm_p[None, :, :]
           + dA[:, :, None] * _bf(wdlo)[None, :, :]
           + dB[:, :, None] * _bf(wdhi)[None, :, :])
    e = _lrelu(pre, alpha)
    e2 = _lrelu(_dot(e.reshape(N * N, 2 * D), W1bd) + b1T, alpha)
    agg_p = jnp.sum(e2.reshape(N, N, 2 * D), axis=1)
    agg = jnp.concatenate([agg_p[:, :D], agg_p[:, D:]], axis=0)
    n = _lrelu(_dot(h_rows, n0a) + _dot(agg, n0b) + nb0, alpha)
    n = _lrelu(_dot(n, n1) + nb1, alpha)
    n = _dot(n, n2) + nb2
    if not last:
        n = _lrelu(n, alpha)
    return n


def _mp_kernel(h_ref,
               wi0, wj0, wdlo0, wdhi0, e0b, w01, b01, na0, nb0_, nbi0, n01,
               nbi01, n02, nbi02,
               wi1, wj1, wdlo1, wdhi1, e1b, w11, b11, na1, nb1_, nbi1, n11,
               nbi11, n12, nbi12,
               out_ref):
    h = h_ref[...].reshape(2 * N, D)
    h1 = _mp_block(h, wi0[...], wj0[...], wdlo0[...], wdhi0[...], e0b[...],
                   w01[...], b01[...], na0[...], nb0_[...], nbi0[...],
                   n01[...], nbi01[...], n02[...], nbi02[...], ALPHA0, False)
    out = _mp_block(h1, wi1[...], wj1[...], wdlo1[...], wdhi1[...], e1b[...],
                    w11[...], b11[...], na1[...], nb1_[...], nbi1[...],
                    n11[...], nbi11[...], n12[...], nbi12[...], ALPHA1, True)
    out_ref[...] = out.reshape(2, N, OUT_PAD)


def kernel(x, W0, b0, ew0_0, eb0_0, ew0_1, eb0_1, nw0_0, nb0_0, nw0_1, nb0_1,
           nw0_2, nb0_2, ew1_0, eb1_0, ew1_1, eb1_1, nw1_0, nb1_0, nw1_1,
           nb1_1, nw1_2, nb1_2):
    H = pl.pallas_call(
        _proj_kernel,
        out_shape=jax.ShapeDtypeStruct((B, N * D), jnp.float32),
    )(x, W0, b0.reshape(1, N * D))
    h0 = H.reshape(B, N, D)

    def split_edge(ew):
        wi, wj, wd = ew[:D], ew[D:2 * D], ew[2 * D:2 * D + 1]
        z = jnp.zeros_like(wd)
        return wi, wj, jnp.concatenate([wd, z], 1), jnp.concatenate([z, wd], 1)

    def bdiag(w):
        z = jnp.zeros_like(w)
        return jnp.concatenate(
            [jnp.concatenate([w, z], 1), jnp.concatenate([z, w], 1)], 0)

    two = lambda v: jnp.concatenate([v, v]).reshape(1, -1)
    row = lambda v: v.reshape(1, -1)

    wi0, wj0, wdlo0, wdhi0 = split_edge(ew0_0)
    wi1, wj1, wdlo1, wdhi1 = split_edge(ew1_0)
    na0, nb0h = nw0_0[:D], nw0_0[D:]
    na1, nb1h = nw1_0[:D], nw1_0[D:]
    n12p = jnp.pad(nw1_2, ((0, 0), (0, OUT_PAD - OUT)))
    nbi12p = jnp.pad(nb1_2, (0, OUT_PAD - OUT)).reshape(1, OUT_PAD)

    weights = (
        wi0, wj0, wdlo0, wdhi0, two(eb0_0), bdiag(ew0_1), two(eb0_1),
        na0, nb0h, row(nb0_0), nw0_1, row(nb0_1), nw0_2, row(nb0_2),
        wi1, wj1, wdlo1, wdhi1, two(eb1_0), bdiag(ew1_1), two(eb1_1),
        na1, nb1h, row(nb1_0), nw1_1, row(nb1_1), n12p, nbi12p,
    )

    wspecs = [pl.BlockSpec(w.shape, lambda b, nd=w.ndim: (0,) * nd)
              for w in weights]

    out = pl.pallas_call(
        _mp_kernel,
        grid=(B // 2,),
        in_specs=[pl.BlockSpec((2, N, D), lambda b: (b, 0, 0))] + wspecs,
        out_specs=pl.BlockSpec((2, N, OUT_PAD), lambda b: (b, 0, 0)),
        out_shape=jax.ShapeDtypeStruct((B, N, OUT_PAD), jnp.float32),
    )(h0, *weights)

    return out[:, :, :OUT]
